# Initial kernel scaffold; baseline (speedup 1.0000x reference)
import jax
import jax.numpy as jnp
from jax.experimental import pallas as pl


def kernel(y, cb, cr, y_qt, c_qt):
    raise NotImplementedError("write your pallas kernel here")


# speedup vs baseline: 1.8512x; 1.8512x over previous
import functools
import itertools

import numpy as np
import jax
import jax.numpy as jnp
from jax.experimental import pallas as pl
from jax.experimental.pallas import tpu as pltpu


def _idct_constants():
    alpha = np.array([1.0 / np.sqrt(2)] + [1.0] * 7, dtype=np.float64)
    basis = np.zeros((8, 8, 8, 8), dtype=np.float64)
    for x, y, u, v in itertools.product(range(8), repeat=4):
        basis[x, y, u, v] = (np.cos((2 * u + 1) * x * np.pi / 16)
                             * np.cos((2 * v + 1) * y * np.pi / 16))
    return (np.outer(alpha, alpha).reshape(64).astype(np.float32),
            basis.reshape(64, 64).astype(np.float32))


_ALPHA64_NP, _IDCT64_NP = _idct_constants()


def _decode_kernel(y_ref, cb_ref, cr_ref, wy_ref, wc_ref, o_ref, *,
                   bry, brc, wblk, cblk):
    sy = jnp.dot(y_ref[...], wy_ref[...], preferred_element_type=jnp.float32)
    yimg = (sy.reshape(bry, wblk, 8, 8).transpose(0, 2, 1, 3)
            .reshape(bry * 8, wblk * 8) + 128.0)

    scb = jnp.dot(cb_ref[...], wc_ref[...], preferred_element_type=jnp.float32)
    scr = jnp.dot(cr_ref[...], wc_ref[...], preferred_element_type=jnp.float32)
    cbh = (scb.reshape(brc, cblk, 8, 16).transpose(0, 2, 1, 3)
           .reshape(brc * 8, cblk * 16))
    crh = (scr.reshape(brc, cblk, 8, 16).transpose(0, 2, 1, 3)
           .reshape(brc * 8, cblk * 16))
    cb2 = jnp.repeat(cbh, 2, axis=0)
    cr2 = jnp.repeat(crh, 2, axis=0)

    o_ref[0] = jnp.clip(yimg + 1.402 * cr2, 0.0, 255.0)
    o_ref[1] = jnp.clip(yimg - 0.344136 * cb2 - 0.714136 * cr2, 0.0, 255.0)
    o_ref[2] = jnp.clip(yimg + 1.772 * cb2, 0.0, 255.0)


def _decompress(y, cb, cr, y_qt, c_qt, height, width):
    b, n_y = y.shape[0], y.shape[1]
    n_c = cb.shape[1]

    tile_h = 64 if height % 64 == 0 else height
    ntiles = height // tile_h
    wblk, cblk = width // 8, width // 16
    bry, brc = tile_h // 8, tile_h // 16

    y2 = y.astype(jnp.float32).reshape(b, n_y, 64)
    cb2 = cb.astype(jnp.float32).reshape(b, n_c, 64)
    cr2 = cr.astype(jnp.float32).reshape(b, n_c, 64)

    alpha64 = jnp.asarray(_ALPHA64_NP)
    idct64 = jnp.asarray(_IDCT64_NP)
    qa_y = y_qt.astype(jnp.float32).reshape(64) * alpha64
    wy = 0.25 * qa_y[:, None] * idct64
    qa_c = c_qt.astype(jnp.float32).reshape(64) * alpha64
    wc = 0.25 * qa_c[:, None] * idct64
    wcu = jnp.repeat(wc.reshape(64, 8, 8), 2, axis=2).reshape(64, 128)

    yspec = pl.BlockSpec((None, bry * wblk, 64), lambda bi, i: (bi, i, 0))
    cspec = pl.BlockSpec((None, brc * cblk, 64), lambda bi, i: (bi, i, 0))

    return pl.pallas_call(
        functools.partial(_decode_kernel, bry=bry, brc=brc, wblk=wblk,
                          cblk=cblk),
        out_shape=jax.ShapeDtypeStruct((b, 3, height, width), jnp.float32),
        grid=(b, ntiles),
        in_specs=[
            yspec, cspec, cspec,
            pl.BlockSpec((64, 64), lambda bi, i: (0, 0)),
            pl.BlockSpec((64, 128), lambda bi, i: (0, 0)),
        ],
        out_specs=pl.BlockSpec((None, 3, tile_h, width),
                               lambda bi, i: (bi, 0, i, 0)),
        compiler_params=pltpu.CompilerParams(
            dimension_semantics=("parallel", "parallel")),
    )(y2, cb2, cr2, wy, wcu)


def kernel(y, cb, cr, y_qt, c_qt):
    return _decompress(y, cb, cr, y_qt, c_qt, 512, 512)


# speedup vs baseline: 3.3446x; 1.8067x over previous
import functools

import numpy as np
import jax
import jax.numpy as jnp
from jax.experimental import pallas as pl
from jax.experimental.pallas import tpu as pltpu


def _dct_factors():
    alpha = np.array([1.0 / np.sqrt(2)] + [1.0] * 7, dtype=np.float64)
    k = np.arange(8)
    cos = np.cos((2 * k[None, :] + 1) * k[:, None] * np.pi / 16)
    a = 0.5 * alpha[:, None] * cos
    return a


_A_NP = _dct_factors()


def _dec_kernel(ym_ref, cbm_ref, crm_ref, qy_ref, qc_ref,
                m2y_ref, m1y_ref, m2c_ref, m1c_ref, o_ref):
    cy = ym_ref[...].astype(jnp.float32) * qy_ref[...]
    t = jnp.dot(cy, m1y_ref[...], preferred_element_type=jnp.float32)
    yimg = jnp.dot(m2y_ref[...], t, preferred_element_type=jnp.float32) + 128.0

    ccb = cbm_ref[...].astype(jnp.float32) * qc_ref[...]
    ccr = crm_ref[...].astype(jnp.float32) * qc_ref[...]
    tcb = jnp.dot(m2c_ref[...], ccb, preferred_element_type=jnp.float32)
    tcr = jnp.dot(m2c_ref[...], ccr, preferred_element_type=jnp.float32)
    cb2 = jnp.dot(tcb, m1c_ref[...], preferred_element_type=jnp.float32)
    cr2 = jnp.dot(tcr, m1c_ref[...], preferred_element_type=jnp.float32)

    o_ref[0] = jnp.clip(yimg + 1.402 * cr2, 0.0, 255.0)
    o_ref[1] = jnp.clip(yimg - 0.344136 * cb2 - 0.714136 * cr2, 0.0, 255.0)
    o_ref[2] = jnp.clip(yimg + 1.772 * cb2, 0.0, 255.0)


def _merge_layout(x, b, nbr, nbc):
    x = x.astype(jnp.bfloat16).reshape(b, nbr, nbc, 8, 8)
    return jnp.transpose(x, (0, 1, 3, 2, 4)).reshape(b, nbr * 8, nbc * 8)


def _decompress(y, cb, cr, y_qt, c_qt, height, width):
    b = y.shape[0]
    tile_h = 64 if height % 64 == 0 else height
    ntiles = height // tile_h
    hw, cw = width, width // 2

    ym = _merge_layout(y, b, height // 8, width // 8)
    cbm = _merge_layout(cb, b, height // 16, width // 16)
    crm = _merge_layout(cr, b, height // 16, width // 16)

    a = _A_NP
    m1y = jnp.asarray(np.kron(np.eye(width // 8), a), dtype=jnp.float32)
    m2y = jnp.asarray(np.kron(np.eye(tile_h // 8), a.T), dtype=jnp.float32)
    a_up_cols = np.repeat(a, 2, axis=1)
    a_up_rows = np.repeat(a.T, 2, axis=0)
    m1c = jnp.asarray(np.kron(np.eye(width // 16), a_up_cols),
                      dtype=jnp.float32)
    m2c = jnp.asarray(np.kron(np.eye(tile_h // 16), a_up_rows),
                      dtype=jnp.float32)

    qy = jnp.tile(y_qt.astype(jnp.float32), (tile_h // 8, width // 8))
    qc = jnp.tile(c_qt.astype(jnp.float32), (tile_h // 16, width // 16))

    cst = lambda r, c: pl.BlockSpec((r, c), lambda bi, i: (0, 0))
    return pl.pallas_call(
        _dec_kernel,
        out_shape=jax.ShapeDtypeStruct((b, 3, height, width), jnp.float32),
        grid=(b, ntiles),
        in_specs=[
            pl.BlockSpec((None, tile_h, hw), lambda bi, i: (bi, i, 0)),
            pl.BlockSpec((None, tile_h // 2, cw), lambda bi, i: (bi, i, 0)),
            pl.BlockSpec((None, tile_h // 2, cw), lambda bi, i: (bi, i, 0)),
            cst(tile_h, hw), cst(tile_h // 2, cw),
            cst(tile_h, tile_h), cst(hw, hw),
            cst(tile_h, tile_h // 2), cst(cw, hw),
        ],
        out_specs=pl.BlockSpec((None, 3, tile_h, width),
                               lambda bi, i: (bi, 0, i, 0)),
        compiler_params=pltpu.CompilerParams(
            dimension_semantics=("parallel", "parallel")),
    )(ym, cbm, crm, qy, qc, m2y, m1y, m2c, m1c)


def kernel(y, cb, cr, y_qt, c_qt):
    return _decompress(y, cb, cr, y_qt, c_qt, 512, 512)
